# baseline (speedup 1.0000x reference)
import functools

import jax
import jax.numpy as jnp
from jax import lax
from jax.experimental import pallas as pl
from jax.experimental.pallas import tpu as pltpu
from jax.experimental.pallas import tpu_sc as plsc

NODES = 120
E = 50000
NP = 128
H = 128
NC = 2
NS = 16
L = 16
EPT = 3136
EPAD = EPT * NS
GBLK = 4
NBLK = EPT // (L * GBLK)
ACC_R = 256
PADV = 120

f32 = jnp.float32
i32 = jnp.int32


ACC_W = ACC_R * H
CNT_BASE = NP * H


def _sc_edge_body(nch, edim, write_e,
                  src_hbm, dst_hbm, ea_hbm, tab_hbm, scal_hbm, zeros_hbm,
                  *refs):
    if write_e:
        (s_out, e_out, src_v, dst_v, ea0_v, ea1_v, a_v, b_v, q_v, scal_v,
         acc_v, e0_v, e1_v) = refs
    else:
        (s_out, src_v, dst_v, ea0_v, ea1_v, a_v, b_v, q_v, scal_v,
         acc_v) = refs
    ea_vs = [ea0_v, ea1_v]

    c = lax.axis_index("c")
    s = lax.axis_index("s")
    off = s * EPT

    pltpu.sync_copy(src_hbm.at[pl.ds(off, EPT)], src_v)
    pltpu.sync_copy(dst_hbm.at[pl.ds(off, EPT)], dst_v)
    if nch == 1:
        pltpu.sync_copy(ea_hbm.at[pl.ds(off, EPT)], ea0_v)
    else:
        pltpu.sync_copy(ea_hbm.at[pl.ds((c * 2) * EPAD + off, EPT)], ea0_v)
        pltpu.sync_copy(ea_hbm.at[pl.ds((c * 2 + 1) * EPAD + off, EPT)],
                        ea1_v)
    tw = NP * H
    pltpu.sync_copy(tab_hbm.at[pl.ds((c * 3 + 0) * tw, tw)], a_v)
    pltpu.sync_copy(tab_hbm.at[pl.ds((c * 3 + 1) * tw, tw)], b_v)
    pltpu.sync_copy(tab_hbm.at[pl.ds((c * 3 + 2) * tw, tw)], q_v)
    pltpu.sync_copy(scal_hbm.at[c], scal_v)
    pltpu.sync_copy(zeros_hbm, acc_v)

    ones_f = jnp.ones((L,), f32)
    NFC = H // L

    def block(b, _):
        ebase = b * (L * GBLK)
        srcs128, dsts128, cidx, eas = [], [], [], []
        for j in range(GBLK):
            sl = pl.ds(ebase + j * L, L)
            srcs128.append(src_v[sl] * H)
            d128 = dst_v[sl] * H
            dsts128.append(d128)
            cidx.append(d128 + CNT_BASE)
            eas.append([ea_vs[ch][sl] for ch in range(nch)])

        bev = scal_v[6, pl.ds(0, L)]
        init = tuple(jnp.full((L,), bev[k], f32)
                     for j in range(GBLK) for k in range(edim))

        def p1(fo, carry):
            es = list(carry)
            fsl = pl.ds(fo * L, L)
            fbase = jnp.full((L,), fo * L, i32)
            weav = [scal_v[ch, fsl] for ch in range(nch)]
            we2v = [scal_v[2 + k, fsl] for k in range(edim)]
            for fi in range(L):
                fs = fbase + fi
                wea = [weav[ch][fi] for ch in range(nch)]
                we2 = [we2v[k][fi] for k in range(edim)]
                for j in range(GBLK):
                    a = plsc.load_gather(a_v, [srcs128[j] + fs])
                    bb = plsc.load_gather(b_v, [dsts128[j] + fs])
                    g = a + bb
                    for ch in range(nch):
                        g = g + eas[j][ch] * wea[ch]
                    t = jnp.maximum(g, 0.0)
                    for k in range(edim):
                        es[j * edim + k] = es[j * edim + k] + t * we2[k]
            return tuple(es)

        efin = lax.fori_loop(0, NFC, p1, init)

        if write_e:
            for j in range(GBLK):
                sl = pl.ds(ebase + j * L, L)
                e0_v[sl] = efin[j * edim + 0]
                e1_v[sl] = efin[j * edim + 1]

        def p2(fo, carry):
            fsl = pl.ds(fo * L, L)
            fbase = jnp.full((L,), fo * L, i32)
            wbv = [scal_v[4 + k, fsl] for k in range(edim)]
            for fi in range(L):
                fs = fbase + fi
                wb = [wbv[k][fi] for k in range(edim)]
                for j in range(GBLK):
                    iq = dsts128[j] + fs
                    q = plsc.load_gather(q_v, [iq])
                    h = q
                    for k in range(edim):
                        h = h + efin[j * edim + k] * wb[k]
                    sres = jnp.maximum(h, 0.0)
                    plsc.addupdate_scatter(acc_v, [iq], sres)
            return carry

        lax.fori_loop(0, NFC, p2, 0)

        for j in range(GBLK):
            plsc.addupdate_scatter(acc_v, [cidx[j]], ones_f)
        return 0

    lax.fori_loop(0, NBLK, block, 0)

    pltpu.sync_copy(acc_v, s_out.at[pl.ds((c * NS + s) * ACC_W, ACC_W)])

    if write_e:
        pltpu.sync_copy(e0_v, e_out.at[pl.ds((c * 2) * EPAD + off, EPT)])
        pltpu.sync_copy(e1_v, e_out.at[pl.ds((c * 2 + 1) * EPAD + off, EPT)])


def _make_sc_kernel(nch, edim, write_e):
    mesh = plsc.VectorSubcoreMesh(core_axis_name="c", subcore_axis_name="s")
    out_type = [jax.ShapeDtypeStruct((2 * NS * ACC_W,), f32)]
    if write_e:
        out_type.append(jax.ShapeDtypeStruct((4 * EPAD,), f32))
    scratch = [
        pltpu.VMEM((EPT,), i32),
        pltpu.VMEM((EPT,), i32),
        pltpu.VMEM((EPT,), f32),
        pltpu.VMEM((EPT,), f32),
        pltpu.VMEM((NP * H,), f32),
        pltpu.VMEM((NP * H,), f32),
        pltpu.VMEM((NP * H,), f32),
        pltpu.VMEM((8, H), f32),
        pltpu.VMEM((ACC_W,), f32),
    ]
    if write_e:
        scratch.append(pltpu.VMEM((EPT,), f32))
        scratch.append(pltpu.VMEM((EPT,), f32))
    body = functools.partial(_sc_edge_body, nch, edim, write_e)
    return pl.kernel(
        body, out_type=out_type, mesh=mesh,
        compiler_params=pltpu.CompilerParams(needs_layout_passes=False),
        scratch_types=scratch)


_sc_layer1 = _make_sc_kernel(nch=1, edim=2, write_e=True)
_sc_layer2 = _make_sc_kernel(nch=2, edim=1, write_e=False)


_ZPAD8 = 8


def _pad_tab(x):
    return jnp.concatenate([x, jnp.zeros((NP - NODES, H), f32)], axis=0)


def _tc1_body(x0_r, u0_r, w1_r, b1_r, w2t_r, b2_r, wm1_r, bm1_r,
              tab_r, scal_r):
    x0 = x0_r[...]
    u0 = u0_r[...]
    z128 = jnp.zeros((H,), f32)
    for c in range(2):
        W1 = w1_r[c]
        A = x0 @ W1[0:5] + u0 @ W1[11:17] + b1_r[c][None, :]
        B = x0 @ W1[5:10]
        Wm1 = wm1_r[c]
        Q = x0 @ Wm1[0:5] + bm1_r[c][None, :]
        tab_r[c, 0] = _pad_tab(A)
        tab_r[c, 1] = _pad_tab(B)
        tab_r[c, 2] = _pad_tab(Q)
        scal_r[c] = jnp.stack([
            W1[10], z128, w2t_r[c][0], w2t_r[c][1], Wm1[5], Wm1[6],
            jnp.concatenate([b2_r[c], jnp.zeros((H - 2,), f32)]), z128])


_tc1 = pl.pallas_call(
    _tc1_body,
    out_shape=[jax.ShapeDtypeStruct((2, 3, NP, H), f32),
               jax.ShapeDtypeStruct((2, 8, H), f32)],
)


def _rnd(x):
    return x.astype(jnp.bfloat16).astype(f32)


def _bdot(a, b):
    return jnp.dot(_rnd(a), _rnd(b), precision=lax.Precision.HIGHEST)


def _sum_parts(s_r, c):
    acc = s_r[c, 0]
    for t in range(1, NS):
        acc = acc + s_r[c, t]
    return acc


def _tc2_body(s1_r, x0_r, u0_r,
              wm2_r, bm2_r, wn1_r, bn1_r, wn2_r, bn2_r,
              wg1_r, bg1_r, wg2_r, bg2_r,
              w1b_r, b1b_r, w2bt_r, b2b_r, wm1b_r, bm1b_r,
              tab_r, scal_r, x1_r):
    x0 = x0_r[...]
    u0 = u0_r[...]
    z128 = jnp.zeros((H,), f32)
    for c in range(2):
        accs = _sum_parts(s1_r, c)
        S = accs[0:NODES]
        cnt = accs[NP:NP + NODES, 0:1]
        agg = S @ wm2_r[c] + cnt * bm2_r[c][None, :]
        aggm = agg / jnp.maximum(cnt, 1.0)
        Wn1 = wn1_r[c]
        h2 = (_bdot(x0, Wn1[0:5]) + _bdot(aggm, Wn1[5:133])
              + _rnd(cnt) * _rnd(Wn1[133])[None, :] + bn1_r[c][None, :])
        x1 = _bdot(jnp.maximum(h2, 0.0), wn2_r[c]) + bn2_r[c][None, :]
        x1_r[c] = x1
        xbar = jnp.mean(x1, axis=0, keepdims=True)
        Wg1 = wg1_r[c]
        hg = u0 @ Wg1[0:6] + xbar @ Wg1[6:16] + bg1_r[c][None, :]
        u1 = jnp.maximum(hg, 0.0) @ wg2_r[c] + bg2_r[c][None, :]
        W1b = w1b_r[c]
        A2 = x1 @ W1b[0:10] + u1 @ W1b[22:34] + b1b_r[c][None, :]
        B2 = x1 @ W1b[10:20]
        Wm1b = wm1b_r[c]
        Q2 = x1 @ Wm1b[0:10] + bm1b_r[c][None, :]
        tab_r[c, 0] = _pad_tab(A2)
        tab_r[c, 1] = _pad_tab(B2)
        tab_r[c, 2] = _pad_tab(Q2)
        scal_r[c] = jnp.stack([
            W1b[20], W1b[21], w2bt_r[c][0], z128, Wm1b[10], z128,
            jnp.concatenate([b2b_r[c], jnp.zeros((H - 1,), f32)]), z128])


_tc2 = pl.pallas_call(
    _tc2_body,
    out_shape=[jax.ShapeDtypeStruct((2, 3, NP, H), f32),
               jax.ShapeDtypeStruct((2, 8, H), f32),
               jax.ShapeDtypeStruct((2, NODES, 10), f32)],
)


def _tc3_body(s2_r, x1_r, wm2_r, bm2_r, wn1_r, bn1_r, wn2t_r, bn2_r, out_r):
    for c in range(2):
        accs = _sum_parts(s2_r, c)
        S = accs[0:NODES]
        cnt = accs[NP:NP + NODES, 0:1]
        agg = S @ wm2_r[c] + cnt * bm2_r[c][None, :]
        aggm = agg / jnp.maximum(cnt, 1.0)
        Wn1 = wn1_r[c]
        h2 = (_bdot(x1_r[c], Wn1[0:10]) + _bdot(aggm, Wn1[10:138])
              + _rnd(cnt) * _rnd(Wn1[138])[None, :] + bn1_r[c][None, :])
        r = jnp.maximum(h2, 0.0)
        xfT = lax.dot_general(_rnd(wn2t_r[c]), _rnd(r),
                              (((1,), (1,)), ((), ())),
                              precision=lax.Precision.HIGHEST)
        out_r[c] = xfT[0] + bn2_r[c, 0]


_tc3 = pl.pallas_call(
    _tc3_body,
    out_shape=jax.ShapeDtypeStruct((2, NODES), f32),
)


def _stack(params, block_a, block_b, group, idx, part):
    wa = params[block_a][group][idx][part]
    wb = params[block_b][group][idx][part]
    return jnp.stack([wa, wb])


def kernel(features, params):
    feats = features[0]
    base = 5 * NODES + 6

    cap = feats[NODES:2 * NODES]
    deg = feats[0:NODES]
    inc = feats[2 * NODES:3 * NODES]
    outg = feats[3 * NODES:4 * NODES]
    tot = feats[4 * NODES:5 * NODES]
    u0 = feats[5 * NODES:base][None, :]
    x0 = jnp.stack([cap, deg, inc, outg, tot], axis=1)

    ea = feats[base:base + E]
    src = feats[base + E:base + 2 * E].astype(i32)
    dst = feats[base + 2 * E:base + 3 * E].astype(i32)

    pad_i = jnp.full((EPAD - E,), PADV, i32)
    src_p = jnp.concatenate([src, pad_i])
    dst_p = jnp.concatenate([dst, pad_i])
    ea_p = jnp.concatenate([ea, jnp.zeros((EPAD - E,), f32)])

    zeros_acc = jnp.zeros((ACC_W,), f32)

    st = functools.partial(_stack, params)

    w2_1 = st('p1', 'v1', 'edge', 1, 0)
    tab1, scal1 = _tc1(
        x0, u0,
        st('p1', 'v1', 'edge', 0, 0), st('p1', 'v1', 'edge', 0, 1),
        jnp.transpose(w2_1, (0, 2, 1)), st('p1', 'v1', 'edge', 1, 1),
        st('p1', 'v1', 'node_mlp1', 0, 0), st('p1', 'v1', 'node_mlp1', 0, 1),
    )

    s1, e1 = _sc_layer1(src_p, dst_p, ea_p, tab1.reshape(-1), scal1,
                        zeros_acc)
    s1 = s1.reshape(2, NS, ACC_R, H)

    w2_2 = st('p2', 'v2', 'edge', 1, 0)
    tab2, scal2, x1 = _tc2(
        s1, x0, u0,
        st('p1', 'v1', 'node_mlp1', 1, 0), st('p1', 'v1', 'node_mlp1', 1, 1),
        st('p1', 'v1', 'node_mlp2', 0, 0), st('p1', 'v1', 'node_mlp2', 0, 1),
        st('p1', 'v1', 'node_mlp2', 1, 0), st('p1', 'v1', 'node_mlp2', 1, 1),
        st('p1', 'v1', 'global', 0, 0), st('p1', 'v1', 'global', 0, 1),
        st('p1', 'v1', 'global', 1, 0), st('p1', 'v1', 'global', 1, 1),
        st('p2', 'v2', 'edge', 0, 0), st('p2', 'v2', 'edge', 0, 1),
        jnp.transpose(w2_2, (0, 2, 1)), st('p2', 'v2', 'edge', 1, 1),
        st('p2', 'v2', 'node_mlp1', 0, 0), st('p2', 'v2', 'node_mlp1', 0, 1),
    )

    (s2,) = _sc_layer2(src_p, dst_p, e1, tab2.reshape(-1), scal2, zeros_acc)
    s2 = s2.reshape(2, NS, ACC_R, H)

    wn2_2 = st('p2', 'v2', 'node_mlp2', 1, 0)
    out = _tc3(
        s2, x1,
        st('p2', 'v2', 'node_mlp1', 1, 0), st('p2', 'v2', 'node_mlp1', 1, 1),
        st('p2', 'v2', 'node_mlp2', 0, 0), st('p2', 'v2', 'node_mlp2', 0, 1),
        jnp.transpose(wn2_2, (0, 2, 1)), st('p2', 'v2', 'node_mlp2', 1, 1),
    )
    return out[0:1], out[1:2]


# speedup vs baseline: 1.9723x; 1.9723x over previous
import functools

import jax
import jax.numpy as jnp
from jax import lax
from jax.experimental import pallas as pl
from jax.experimental.pallas import tpu as pltpu
from jax.experimental.pallas import tpu_sc as plsc

NODES = 120
E = 50000
NP = 128
H = 128
NC = 2
NS = 16
L = 16
EPT = 3136
EPAD = EPT * NS
GBLK = 4
NBLK = EPT // (L * GBLK)
ACC_R = 256
PADV = 120

f32 = jnp.float32
i32 = jnp.int32


ACC_W = ACC_R * H
CNT_BASE = NP * H


def _sc_edge_body(nch, edim, write_e,
                  src_hbm, dst_hbm, ea_hbm, tab_hbm, scal_hbm, zeros_hbm,
                  *refs):
    if write_e:
        (s_out, e_out, src_v, dst_v, ea0_v, ea1_v, a_v, b_v, q_v, scal_v,
         acc_v, e0_v, e1_v) = refs
    else:
        (s_out, src_v, dst_v, ea0_v, ea1_v, a_v, b_v, q_v, scal_v,
         acc_v) = refs
    ea_vs = [ea0_v, ea1_v]

    c = lax.axis_index("c")
    s = lax.axis_index("s")
    off = s * EPT

    pltpu.sync_copy(src_hbm.at[pl.ds(off, EPT)], src_v)
    pltpu.sync_copy(dst_hbm.at[pl.ds(off, EPT)], dst_v)
    if nch == 1:
        pltpu.sync_copy(ea_hbm.at[pl.ds(off, EPT)], ea0_v)
    else:
        pltpu.sync_copy(ea_hbm.at[pl.ds((c * 2) * EPAD + off, EPT)], ea0_v)
        pltpu.sync_copy(ea_hbm.at[pl.ds((c * 2 + 1) * EPAD + off, EPT)],
                        ea1_v)
    tw = NP * H
    pltpu.sync_copy(tab_hbm.at[pl.ds((c * 3 + 0) * tw, tw)], a_v)
    pltpu.sync_copy(tab_hbm.at[pl.ds((c * 3 + 1) * tw, tw)], b_v)
    pltpu.sync_copy(tab_hbm.at[pl.ds((c * 3 + 2) * tw, tw)], q_v)
    pltpu.sync_copy(scal_hbm.at[c], scal_v)
    pltpu.sync_copy(zeros_hbm, acc_v)

    ones_f = jnp.ones((L,), f32)
    NFC = H // L

    def block(b, _):
        ebase = b * (L * GBLK)
        srcs, dsts, cidx, eas = [], [], [], []
        for j in range(GBLK):
            sl = pl.ds(ebase + j * L, L)
            srcs.append(src_v[sl])
            dj = dst_v[sl]
            dsts.append(dj)
            cidx.append(dj * NP + CNT_BASE)
            eas.append([ea_vs[ch][sl] for ch in range(nch)])

        bev = scal_v[6, pl.ds(0, L)]
        init = tuple(jnp.full((L,), bev[k], f32)
                     for j in range(GBLK) for k in range(edim))

        def p1(fo, carry):
            es = list(carry)
            fsl = pl.ds(fo * L, L)
            fbase = jnp.full((L,), fo * L * NP, i32)
            weav = [scal_v[ch, fsl] for ch in range(nch)]
            we2v = [scal_v[2 + k, fsl] for k in range(edim)]
            for fi in range(L):
                fs = fbase + (fi * NP)
                wea = [weav[ch][fi] for ch in range(nch)]
                we2 = [we2v[k][fi] for k in range(edim)]
                for j in range(GBLK):
                    a = plsc.load_gather(a_v, [srcs[j] + fs])
                    bb = plsc.load_gather(b_v, [dsts[j] + fs])
                    g = a + bb
                    for ch in range(nch):
                        g = g + eas[j][ch] * wea[ch]
                    t = jnp.maximum(g, 0.0)
                    for k in range(edim):
                        es[j * edim + k] = es[j * edim + k] + t * we2[k]
            return tuple(es)

        efin = lax.fori_loop(0, NFC, p1, init)

        if write_e:
            for j in range(GBLK):
                sl = pl.ds(ebase + j * L, L)
                e0_v[sl] = efin[j * edim + 0]
                e1_v[sl] = efin[j * edim + 1]

        def p2(fo, carry):
            fsl = pl.ds(fo * L, L)
            fbase = jnp.full((L,), fo * L * NP, i32)
            wbv = [scal_v[4 + k, fsl] for k in range(edim)]
            for fi in range(L):
                fs = fbase + (fi * NP)
                wb = [wbv[k][fi] for k in range(edim)]
                for j in range(GBLK):
                    iq = dsts[j] + fs
                    q = plsc.load_gather(q_v, [iq])
                    h = q
                    for k in range(edim):
                        h = h + efin[j * edim + k] * wb[k]
                    sres = jnp.maximum(h, 0.0)
                    plsc.addupdate_scatter(acc_v, [iq], sres)
            return carry

        lax.fori_loop(0, NFC, p2, 0)

        for j in range(GBLK):
            plsc.addupdate_scatter(acc_v, [cidx[j]], ones_f)
        return 0

    lax.fori_loop(0, NBLK, block, 0)

    pltpu.sync_copy(acc_v, s_out.at[pl.ds((c * NS + s) * ACC_W, ACC_W)])

    if write_e:
        pltpu.sync_copy(e0_v, e_out.at[pl.ds((c * 2) * EPAD + off, EPT)])
        pltpu.sync_copy(e1_v, e_out.at[pl.ds((c * 2 + 1) * EPAD + off, EPT)])


def _make_sc_kernel(nch, edim, write_e):
    mesh = plsc.VectorSubcoreMesh(core_axis_name="c", subcore_axis_name="s")
    out_type = [jax.ShapeDtypeStruct((2 * NS * ACC_W,), f32)]
    if write_e:
        out_type.append(jax.ShapeDtypeStruct((4 * EPAD,), f32))
    scratch = [
        pltpu.VMEM((EPT,), i32),
        pltpu.VMEM((EPT,), i32),
        pltpu.VMEM((EPT,), f32),
        pltpu.VMEM((EPT,), f32),
        pltpu.VMEM((NP * H,), f32),
        pltpu.VMEM((NP * H,), f32),
        pltpu.VMEM((NP * H,), f32),
        pltpu.VMEM((8, H), f32),
        pltpu.VMEM((ACC_W,), f32),
    ]
    if write_e:
        scratch.append(pltpu.VMEM((EPT,), f32))
        scratch.append(pltpu.VMEM((EPT,), f32))
    body = functools.partial(_sc_edge_body, nch, edim, write_e)
    return pl.kernel(
        body, out_type=out_type, mesh=mesh,
        compiler_params=pltpu.CompilerParams(needs_layout_passes=False),
        scratch_types=scratch)


_sc_layer1 = _make_sc_kernel(nch=1, edim=2, write_e=True)
_sc_layer2 = _make_sc_kernel(nch=2, edim=1, write_e=False)


def _tdot(w, xp):
    return lax.dot_general(w, xp, (((0,), (1,)), ((), ())))


def _col(v):
    return lax.dot_general(v, jnp.ones((1, 1), f32), (((0,), (0,)), ((), ())))


def _tc1_body(x0_r, u0_r, w1_r, b1_r, w2t_r, b2_r, wm1_r, bm1_r,
              tab_r, scal_r):
    x0 = x0_r[...]
    u0 = u0_r[...]
    z128 = jnp.zeros((H,), f32)
    for c in range(2):
        W1 = w1_r[c]
        Wm1 = wm1_r[c]
        tab_r[c, 0] = (_tdot(W1[0:5], x0)
                       + _col(u0 @ W1[11:17] + b1_r[c][None, :]))
        tab_r[c, 1] = _tdot(W1[5:10], x0)
        tab_r[c, 2] = _tdot(Wm1[0:5], x0) + _col(bm1_r[c][None, :])
        scal_r[c] = jnp.stack([
            W1[10], z128, w2t_r[c][0], w2t_r[c][1], Wm1[5], Wm1[6],
            jnp.concatenate([b2_r[c], jnp.zeros((H - 2,), f32)]), z128])


_tc1 = pl.pallas_call(
    _tc1_body,
    out_shape=[jax.ShapeDtypeStruct((2, 3, NP, H), f32),
               jax.ShapeDtypeStruct((2, 8, H), f32)],
)


def _rnd(x):
    return x.astype(jnp.bfloat16).astype(f32)


def _bdot(a, b):
    return jnp.dot(_rnd(a), _rnd(b), precision=lax.Precision.HIGHEST)


def _sum_parts(s_r, c):
    acc = s_r[c, 0]
    for t in range(1, NS):
        acc = acc + s_r[c, t]
    return acc


def _tc2_body(s1_r, x0_r, u0_r,
              wm2_r, bm2_r, wn1_r, bn1_r, wn2_r, bn2_r,
              wg1_r, bg1_r, wg2_r, bg2_r,
              w1b_r, b1b_r, w2bt_r, b2b_r, wm1b_r, bm1b_r,
              tab_r, scal_r, x1_r):
    x0 = x0_r[...]
    u0 = u0_r[...]
    z128 = jnp.zeros((H,), f32)
    for c in range(2):
        accs = _sum_parts(s1_r, c)
        agg = (lax.dot_general(accs[0:NP], wm2_r[c], (((0,), (0,)), ((), ())))
               [0:NODES] + accs[NP:NP + NODES, 0:1] * bm2_r[c][None, :])
        cnt = accs[NP:NP + NODES, 0:1]
        aggm = agg / jnp.maximum(cnt, 1.0)
        Wn1 = wn1_r[c]
        h2 = (_bdot(x0[0:NODES], Wn1[0:5]) + _bdot(aggm, Wn1[5:133])
              + _rnd(cnt) * _rnd(Wn1[133])[None, :] + bn1_r[c][None, :])
        x1 = _bdot(jnp.maximum(h2, 0.0), wn2_r[c]) + bn2_r[c][None, :]
        x1p = jnp.concatenate([x1, jnp.zeros((NP - NODES, 10), f32)], axis=0)
        x1_r[c] = x1p
        xbar = jnp.mean(x1, axis=0, keepdims=True)
        Wg1 = wg1_r[c]
        hg = u0 @ Wg1[0:6] + xbar @ Wg1[6:16] + bg1_r[c][None, :]
        u1 = jnp.maximum(hg, 0.0) @ wg2_r[c] + bg2_r[c][None, :]
        W1b = w1b_r[c]
        Wm1b = wm1b_r[c]
        tab_r[c, 0] = (_tdot(W1b[0:10], x1p)
                       + _col(u1 @ W1b[22:34] + b1b_r[c][None, :]))
        tab_r[c, 1] = _tdot(W1b[10:20], x1p)
        tab_r[c, 2] = _tdot(Wm1b[0:10], x1p) + _col(bm1b_r[c][None, :])
        scal_r[c] = jnp.stack([
            W1b[20], W1b[21], w2bt_r[c][0], z128, Wm1b[10], z128,
            jnp.concatenate([b2b_r[c], jnp.zeros((H - 1,), f32)]), z128])


_tc2 = pl.pallas_call(
    _tc2_body,
    out_shape=[jax.ShapeDtypeStruct((2, 3, NP, H), f32),
               jax.ShapeDtypeStruct((2, 8, H), f32),
               jax.ShapeDtypeStruct((2, NP, 10), f32)],
)


def _tc3_body(s2_r, x1_r, wm2_r, bm2_r, wn1_r, bn1_r, wn2t_r, bn2_r, out_r):
    for c in range(2):
        accs = _sum_parts(s2_r, c)
        agg = (lax.dot_general(accs[0:NP], wm2_r[c], (((0,), (0,)), ((), ())))
               [0:NODES] + accs[NP:NP + NODES, 0:1] * bm2_r[c][None, :])
        cnt = accs[NP:NP + NODES, 0:1]
        aggm = agg / jnp.maximum(cnt, 1.0)
        Wn1 = wn1_r[c]
        h2 = (_bdot(x1_r[c][0:NODES], Wn1[0:10]) + _bdot(aggm, Wn1[10:138])
              + _rnd(cnt) * _rnd(Wn1[138])[None, :] + bn1_r[c][None, :])
        r = jnp.maximum(h2, 0.0)
        xfT = lax.dot_general(_rnd(wn2t_r[c]), _rnd(r),
                              (((1,), (1,)), ((), ())),
                              precision=lax.Precision.HIGHEST)
        out_r[c] = xfT[0] + bn2_r[c, 0]


_tc3 = pl.pallas_call(
    _tc3_body,
    out_shape=jax.ShapeDtypeStruct((2, NODES), f32),
)


def _stack(params, block_a, block_b, group, idx, part):
    wa = params[block_a][group][idx][part]
    wb = params[block_b][group][idx][part]
    return jnp.stack([wa, wb])


def kernel(features, params):
    feats = features[0]
    base = 5 * NODES + 6

    cap = feats[NODES:2 * NODES]
    deg = feats[0:NODES]
    inc = feats[2 * NODES:3 * NODES]
    outg = feats[3 * NODES:4 * NODES]
    tot = feats[4 * NODES:5 * NODES]
    u0 = feats[5 * NODES:base][None, :]
    x0 = jnp.stack([cap, deg, inc, outg, tot], axis=1)
    x0 = jnp.concatenate([x0, jnp.zeros((NP - NODES, 5), f32)], axis=0)

    ea = feats[base:base + E]
    src = feats[base + E:base + 2 * E].astype(i32)
    dst = feats[base + 2 * E:base + 3 * E].astype(i32)

    pad_i = jnp.full((EPAD - E,), PADV, i32)
    src_p = jnp.concatenate([src, pad_i])
    dst_p = jnp.concatenate([dst, pad_i])
    ea_p = jnp.concatenate([ea, jnp.zeros((EPAD - E,), f32)])

    zeros_acc = jnp.zeros((ACC_W,), f32)

    st = functools.partial(_stack, params)

    w2_1 = st('p1', 'v1', 'edge', 1, 0)
    tab1, scal1 = _tc1(
        x0, u0,
        st('p1', 'v1', 'edge', 0, 0), st('p1', 'v1', 'edge', 0, 1),
        jnp.transpose(w2_1, (0, 2, 1)), st('p1', 'v1', 'edge', 1, 1),
        st('p1', 'v1', 'node_mlp1', 0, 0), st('p1', 'v1', 'node_mlp1', 0, 1),
    )

    s1, e1 = _sc_layer1(src_p, dst_p, ea_p, tab1.reshape(-1), scal1,
                        zeros_acc)
    s1 = s1.reshape(2, NS, ACC_R, H)

    w2_2 = st('p2', 'v2', 'edge', 1, 0)
    tab2, scal2, x1 = _tc2(
        s1, x0, u0,
        st('p1', 'v1', 'node_mlp1', 1, 0), st('p1', 'v1', 'node_mlp1', 1, 1),
        st('p1', 'v1', 'node_mlp2', 0, 0), st('p1', 'v1', 'node_mlp2', 0, 1),
        st('p1', 'v1', 'node_mlp2', 1, 0), st('p1', 'v1', 'node_mlp2', 1, 1),
        st('p1', 'v1', 'global', 0, 0), st('p1', 'v1', 'global', 0, 1),
        st('p1', 'v1', 'global', 1, 0), st('p1', 'v1', 'global', 1, 1),
        st('p2', 'v2', 'edge', 0, 0), st('p2', 'v2', 'edge', 0, 1),
        jnp.transpose(w2_2, (0, 2, 1)), st('p2', 'v2', 'edge', 1, 1),
        st('p2', 'v2', 'node_mlp1', 0, 0), st('p2', 'v2', 'node_mlp1', 0, 1),
    )

    (s2,) = _sc_layer2(src_p, dst_p, e1, tab2.reshape(-1), scal2, zeros_acc)
    s2 = s2.reshape(2, NS, ACC_R, H)

    wn2_2 = st('p2', 'v2', 'node_mlp2', 1, 0)
    out = _tc3(
        s2, x1,
        st('p2', 'v2', 'node_mlp1', 1, 0), st('p2', 'v2', 'node_mlp1', 1, 1),
        st('p2', 'v2', 'node_mlp2', 0, 0), st('p2', 'v2', 'node_mlp2', 0, 1),
        jnp.transpose(wn2_2, (0, 2, 1)), st('p2', 'v2', 'node_mlp2', 1, 1),
    )
    return out[0:1], out[1:2]


# speedup vs baseline: 2.1198x; 1.0748x over previous
import functools

import jax
import jax.numpy as jnp
from jax import lax
from jax.experimental import pallas as pl
from jax.experimental.pallas import tpu as pltpu
from jax.experimental.pallas import tpu_sc as plsc

NODES = 120
E = 50000
NP = 128
H = 128
NC = 2
NS = 16
L = 16
EPT = 3136
EPAD = EPT * NS
GBLK = 4
NBLK = EPT // (L * GBLK)
ACC_R = 256
PADV = 120

f32 = jnp.float32
i32 = jnp.int32


ACC_W = ACC_R * H
SCALW = 8 * H * L
CNT_BASE = NP * H


def _sc_edge_body(nch, edim, write_e,
                  src_hbm, dst_hbm, ea_hbm, tab_hbm, scal_hbm, zeros_hbm,
                  *refs):
    if write_e:
        (s_out, e_out, src_v, dst_v, ea0_v, ea1_v, a_v, b_v, q_v, scal_v,
         acc_v, e0_v, e1_v) = refs
    else:
        (s_out, src_v, dst_v, ea0_v, ea1_v, a_v, b_v, q_v, scal_v,
         acc_v) = refs
    ea_vs = [ea0_v, ea1_v]

    c = lax.axis_index("c")
    s = lax.axis_index("s")
    off = s * EPT

    pltpu.sync_copy(src_hbm.at[pl.ds(off, EPT)], src_v)
    pltpu.sync_copy(dst_hbm.at[pl.ds(off, EPT)], dst_v)
    if nch == 1:
        pltpu.sync_copy(ea_hbm.at[pl.ds(off, EPT)], ea0_v)
    else:
        pltpu.sync_copy(ea_hbm.at[pl.ds((c * 2) * EPAD + off, EPT)], ea0_v)
        pltpu.sync_copy(ea_hbm.at[pl.ds((c * 2 + 1) * EPAD + off, EPT)],
                        ea1_v)
    tw = NP * H
    pltpu.sync_copy(tab_hbm.at[pl.ds((c * 3 + 0) * tw, tw)], a_v)
    pltpu.sync_copy(tab_hbm.at[pl.ds((c * 3 + 1) * tw, tw)], b_v)
    pltpu.sync_copy(tab_hbm.at[pl.ds((c * 3 + 2) * tw, tw)], q_v)
    pltpu.sync_copy(scal_hbm.at[pl.ds(c * SCALW, SCALW)], scal_v)
    pltpu.sync_copy(zeros_hbm, acc_v)

    ones_f = jnp.ones((L,), f32)
    NFC = H // L

    def block(b, _):
        ebase = b * (L * GBLK)
        srcs, dsts, cidx, eas = [], [], [], []
        for j in range(GBLK):
            sl = pl.ds(ebase + j * L, L)
            srcs.append(src_v[sl])
            dj = dst_v[sl]
            dsts.append(dj)
            cidx.append(dj * NP + CNT_BASE)
            eas.append([ea_vs[ch][sl] for ch in range(nch)])

        init = tuple(scal_v[pl.ds((6 * H + k) * L, L)]
                     for j in range(GBLK) for k in range(edim))

        def p1(fo, carry):
            es = list(carry)
            fbase = jnp.full((L,), fo * L * NP, i32)
            for fi in range(L):
                fs = fbase + (fi * NP)
                fw = (fo * L + fi) * L
                wea = [scal_v[pl.ds(ch * H * L + fw, L)]
                       for ch in range(nch)]
                we2 = [scal_v[pl.ds((2 + k) * H * L + fw, L)]
                       for k in range(edim)]
                for j in range(GBLK):
                    a = plsc.load_gather(a_v, [srcs[j] + fs])
                    bb = plsc.load_gather(b_v, [dsts[j] + fs])
                    g = a + bb
                    for ch in range(nch):
                        g = g + eas[j][ch] * wea[ch]
                    t = jnp.maximum(g, 0.0)
                    for k in range(edim):
                        es[j * edim + k] = es[j * edim + k] + t * we2[k]
            return tuple(es)

        efin = lax.fori_loop(0, NFC, p1, init)

        if write_e:
            for j in range(GBLK):
                sl = pl.ds(ebase + j * L, L)
                e0_v[sl] = efin[j * edim + 0]
                e1_v[sl] = efin[j * edim + 1]

        def p2(fo, carry):
            fbase = jnp.full((L,), fo * L * NP, i32)
            for fi in range(L):
                fs = fbase + (fi * NP)
                fw = (fo * L + fi) * L
                wb = [scal_v[pl.ds((4 + k) * H * L + fw, L)]
                      for k in range(edim)]
                for j in range(GBLK):
                    iq = dsts[j] + fs
                    q = plsc.load_gather(q_v, [iq])
                    h = q
                    for k in range(edim):
                        h = h + efin[j * edim + k] * wb[k]
                    sres = jnp.maximum(h, 0.0)
                    plsc.addupdate_scatter(acc_v, [iq], sres)
            return carry

        lax.fori_loop(0, NFC, p2, 0)

        for j in range(GBLK):
            plsc.addupdate_scatter(acc_v, [cidx[j]], ones_f)
        return 0

    lax.fori_loop(0, NBLK, block, 0)

    pltpu.sync_copy(acc_v, s_out.at[pl.ds((c * NS + s) * ACC_W, ACC_W)])

    if write_e:
        pltpu.sync_copy(e0_v, e_out.at[pl.ds((c * 2) * EPAD + off, EPT)])
        pltpu.sync_copy(e1_v, e_out.at[pl.ds((c * 2 + 1) * EPAD + off, EPT)])


def _make_sc_kernel(nch, edim, write_e):
    mesh = plsc.VectorSubcoreMesh(core_axis_name="c", subcore_axis_name="s")
    out_type = [jax.ShapeDtypeStruct((2 * NS * ACC_W,), f32)]
    if write_e:
        out_type.append(jax.ShapeDtypeStruct((4 * EPAD,), f32))
    scratch = [
        pltpu.VMEM((EPT,), i32),
        pltpu.VMEM((EPT,), i32),
        pltpu.VMEM((EPT,), f32),
        pltpu.VMEM((EPT,), f32),
        pltpu.VMEM((NP * H,), f32),
        pltpu.VMEM((NP * H,), f32),
        pltpu.VMEM((NP * H,), f32),
        pltpu.VMEM((SCALW,), f32),
        pltpu.VMEM((ACC_W,), f32),
    ]
    if write_e:
        scratch.append(pltpu.VMEM((EPT,), f32))
        scratch.append(pltpu.VMEM((EPT,), f32))
    body = functools.partial(_sc_edge_body, nch, edim, write_e)
    return pl.kernel(
        body, out_type=out_type, mesh=mesh,
        compiler_params=pltpu.CompilerParams(needs_layout_passes=False),
        scratch_types=scratch)


_sc_layer1 = _make_sc_kernel(nch=1, edim=2, write_e=True)
_sc_layer2 = _make_sc_kernel(nch=2, edim=1, write_e=False)


def _tdot(w, xp):
    return lax.dot_general(w, xp, (((0,), (1,)), ((), ())))


def _col(v):
    return lax.dot_general(v, jnp.ones((1, 1), f32), (((0,), (0,)), ((), ())))


def _tc1_body(x0_r, u0_r, w1_r, b1_r, w2t_r, b2_r, wm1_r, bm1_r,
              tab_r, scal_r):
    x0 = x0_r[...]
    u0 = u0_r[...]
    z128 = jnp.zeros((H,), f32)
    for c in range(2):
        W1 = w1_r[c]
        Wm1 = wm1_r[c]
        tab_r[c, 0] = (_tdot(W1[0:5], x0)
                       + _col(u0 @ W1[11:17] + b1_r[c][None, :]))
        tab_r[c, 1] = _tdot(W1[5:10], x0)
        tab_r[c, 2] = _tdot(Wm1[0:5], x0) + _col(bm1_r[c][None, :])
        scal_r[c] = jnp.stack([
            W1[10], z128, w2t_r[c][0], w2t_r[c][1], Wm1[5], Wm1[6],
            jnp.concatenate([b2_r[c], jnp.zeros((H - 2,), f32)]), z128])


_tc1 = pl.pallas_call(
    _tc1_body,
    out_shape=[jax.ShapeDtypeStruct((2, 3, NP, H), f32),
               jax.ShapeDtypeStruct((2, 8, H), f32)],
)


def _rnd(x):
    return x.astype(jnp.bfloat16).astype(f32)


def _bdot(a, b):
    return jnp.dot(_rnd(a), _rnd(b), precision=lax.Precision.HIGHEST)


def _sum_parts(s_r, c):
    acc = s_r[c, 0]
    for t in range(1, NS):
        acc = acc + s_r[c, t]
    return acc


def _tc2_body(s1_r, x0_r, u0_r,
              wm2_r, bm2_r, wn1_r, bn1_r, wn2_r, bn2_r,
              wg1_r, bg1_r, wg2_r, bg2_r,
              w1b_r, b1b_r, w2bt_r, b2b_r, wm1b_r, bm1b_r,
              tab_r, scal_r, x1_r):
    x0 = x0_r[...]
    u0 = u0_r[...]
    z128 = jnp.zeros((H,), f32)
    for c in range(2):
        accs = _sum_parts(s1_r, c)
        agg = (lax.dot_general(accs[0:NP], wm2_r[c], (((0,), (0,)), ((), ())))
               [0:NODES] + accs[NP:NP + NODES, 0:1] * bm2_r[c][None, :])
        cnt = accs[NP:NP + NODES, 0:1]
        aggm = agg / jnp.maximum(cnt, 1.0)
        Wn1 = wn1_r[c]
        h2 = (_bdot(x0[0:NODES], Wn1[0:5]) + _bdot(aggm, Wn1[5:133])
              + _rnd(cnt) * _rnd(Wn1[133])[None, :] + bn1_r[c][None, :])
        x1 = _bdot(jnp.maximum(h2, 0.0), wn2_r[c]) + bn2_r[c][None, :]
        x1p = jnp.concatenate([x1, jnp.zeros((NP - NODES, 10), f32)], axis=0)
        x1_r[c] = x1p
        xbar = jnp.mean(x1, axis=0, keepdims=True)
        Wg1 = wg1_r[c]
        hg = u0 @ Wg1[0:6] + xbar @ Wg1[6:16] + bg1_r[c][None, :]
        u1 = jnp.maximum(hg, 0.0) @ wg2_r[c] + bg2_r[c][None, :]
        W1b = w1b_r[c]
        Wm1b = wm1b_r[c]
        tab_r[c, 0] = (_tdot(W1b[0:10], x1p)
                       + _col(u1 @ W1b[22:34] + b1b_r[c][None, :]))
        tab_r[c, 1] = _tdot(W1b[10:20], x1p)
        tab_r[c, 2] = _tdot(Wm1b[0:10], x1p) + _col(bm1b_r[c][None, :])
        scal_r[c] = jnp.stack([
            W1b[20], W1b[21], w2bt_r[c][0], z128, Wm1b[10], z128,
            jnp.concatenate([b2b_r[c], jnp.zeros((H - 1,), f32)]), z128])


_tc2 = pl.pallas_call(
    _tc2_body,
    out_shape=[jax.ShapeDtypeStruct((2, 3, NP, H), f32),
               jax.ShapeDtypeStruct((2, 8, H), f32),
               jax.ShapeDtypeStruct((2, NP, 10), f32)],
)


def _tc3_body(s2_r, x1_r, wm2_r, bm2_r, wn1_r, bn1_r, wn2t_r, bn2_r, out_r):
    for c in range(2):
        accs = _sum_parts(s2_r, c)
        agg = (lax.dot_general(accs[0:NP], wm2_r[c], (((0,), (0,)), ((), ())))
               [0:NODES] + accs[NP:NP + NODES, 0:1] * bm2_r[c][None, :])
        cnt = accs[NP:NP + NODES, 0:1]
        aggm = agg / jnp.maximum(cnt, 1.0)
        Wn1 = wn1_r[c]
        h2 = (_bdot(x1_r[c][0:NODES], Wn1[0:10]) + _bdot(aggm, Wn1[10:138])
              + _rnd(cnt) * _rnd(Wn1[138])[None, :] + bn1_r[c][None, :])
        r = jnp.maximum(h2, 0.0)
        xfT = lax.dot_general(_rnd(wn2t_r[c]), _rnd(r),
                              (((1,), (1,)), ((), ())),
                              precision=lax.Precision.HIGHEST)
        out_r[c] = xfT[0] + bn2_r[c, 0]


_tc3 = pl.pallas_call(
    _tc3_body,
    out_shape=jax.ShapeDtypeStruct((2, NODES), f32),
)


def _stack(params, block_a, block_b, group, idx, part):
    wa = params[block_a][group][idx][part]
    wb = params[block_b][group][idx][part]
    return jnp.stack([wa, wb])


def kernel(features, params):
    feats = features[0]
    base = 5 * NODES + 6

    cap = feats[NODES:2 * NODES]
    deg = feats[0:NODES]
    inc = feats[2 * NODES:3 * NODES]
    outg = feats[3 * NODES:4 * NODES]
    tot = feats[4 * NODES:5 * NODES]
    u0 = feats[5 * NODES:base][None, :]
    x0 = jnp.stack([cap, deg, inc, outg, tot], axis=1)
    x0 = jnp.concatenate([x0, jnp.zeros((NP - NODES, 5), f32)], axis=0)

    ea = feats[base:base + E]
    src = feats[base + E:base + 2 * E].astype(i32)
    dst = feats[base + 2 * E:base + 3 * E].astype(i32)

    pad_i = jnp.full((EPAD - E,), PADV, i32)
    src_p = jnp.concatenate([src, pad_i])
    dst_p = jnp.concatenate([dst, pad_i])
    ea_p = jnp.concatenate([ea, jnp.zeros((EPAD - E,), f32)])

    zeros_acc = jnp.zeros((ACC_W,), f32)

    st = functools.partial(_stack, params)

    w2_1 = st('p1', 'v1', 'edge', 1, 0)
    tab1, scal1 = _tc1(
        x0, u0,
        st('p1', 'v1', 'edge', 0, 0), st('p1', 'v1', 'edge', 0, 1),
        jnp.transpose(w2_1, (0, 2, 1)), st('p1', 'v1', 'edge', 1, 1),
        st('p1', 'v1', 'node_mlp1', 0, 0), st('p1', 'v1', 'node_mlp1', 0, 1),
    )

    scal1b = jnp.broadcast_to(scal1[:, :, :, None],
                              (2, 8, H, L)).reshape(-1)
    s1, e1 = _sc_layer1(src_p, dst_p, ea_p, tab1.reshape(-1), scal1b,
                        zeros_acc)
    s1 = s1.reshape(2, NS, ACC_R, H)

    w2_2 = st('p2', 'v2', 'edge', 1, 0)
    tab2, scal2, x1 = _tc2(
        s1, x0, u0,
        st('p1', 'v1', 'node_mlp1', 1, 0), st('p1', 'v1', 'node_mlp1', 1, 1),
        st('p1', 'v1', 'node_mlp2', 0, 0), st('p1', 'v1', 'node_mlp2', 0, 1),
        st('p1', 'v1', 'node_mlp2', 1, 0), st('p1', 'v1', 'node_mlp2', 1, 1),
        st('p1', 'v1', 'global', 0, 0), st('p1', 'v1', 'global', 0, 1),
        st('p1', 'v1', 'global', 1, 0), st('p1', 'v1', 'global', 1, 1),
        st('p2', 'v2', 'edge', 0, 0), st('p2', 'v2', 'edge', 0, 1),
        jnp.transpose(w2_2, (0, 2, 1)), st('p2', 'v2', 'edge', 1, 1),
        st('p2', 'v2', 'node_mlp1', 0, 0), st('p2', 'v2', 'node_mlp1', 0, 1),
    )

    scal2b = jnp.broadcast_to(scal2[:, :, :, None],
                              (2, 8, H, L)).reshape(-1)
    (s2,) = _sc_layer2(src_p, dst_p, e1, tab2.reshape(-1), scal2b, zeros_acc)
    s2 = s2.reshape(2, NS, ACC_R, H)

    wn2_2 = st('p2', 'v2', 'node_mlp2', 1, 0)
    out = _tc3(
        s2, x1,
        st('p2', 'v2', 'node_mlp1', 1, 0), st('p2', 'v2', 'node_mlp1', 1, 1),
        st('p2', 'v2', 'node_mlp2', 0, 0), st('p2', 'v2', 'node_mlp2', 0, 1),
        jnp.transpose(wn2_2, (0, 2, 1)), st('p2', 'v2', 'node_mlp2', 1, 1),
    )
    return out[0:1], out[1:2]
